# baseline (speedup 1.0000x reference)
import functools

import jax
import jax.numpy as jnp
from jax import lax
from jax.experimental import pallas as pl
from jax.experimental.pallas import tpu as pltpu
from jax.experimental.pallas import tpu_sc as plsc

N = 10000
E = 320000
F_IN = 128
HID = 128
NCLS = 64

NC = 2
NS = 16
NW = NC * NS

CHUNK = 128
CH = 80
IDXW = 40
EPAD = NW * CH * CHUNK
NPAD = 10240
RPT = NPAD // NS
PAD_SPREAD = NPAD - N

_mesh = plsc.VectorSubcoreMesh(core_axis_name="c", subcore_axis_name="s")


@functools.partial(
    pl.kernel,
    out_type=(
        jax.ShapeDtypeStruct((NC, NPAD, 16), jnp.float32),
        jax.ShapeDtypeStruct((NC, NPAD, 16), jnp.float32),
    ),
    mesh=_mesh,
    compiler_params=pltpu.CompilerParams(use_tc_tiling_on_sc=False),
    scratch_types=[
        pltpu.VMEM((CH, CHUNK), jnp.int32),
        pltpu.VMEM((CH, CHUNK), jnp.int32),
        pltpu.VMEM((CHUNK, 16), jnp.float32),
        pltpu.VMEM_SHARED((NPAD, 16), jnp.float32),
        pltpu.VMEM_SHARED((NPAD, 16), jnp.float32),
        [pltpu.SemaphoreType.DMA] * 4,
    ],
)
def _deg_kernel(src_hbm, dst_hbm, ones_hbm, zeros_hbm, od_out, id_out,
                sidx, didx, ones_v, od_sh, id_sh, dsem):
    c = lax.axis_index("c")
    s = lax.axis_index("s")
    wid = c * NS + s
    pltpu.sync_copy(src_hbm.at[wid], sidx)
    pltpu.sync_copy(dst_hbm.at[wid], didx)
    pltpu.sync_copy(ones_hbm, ones_v)
    pltpu.sync_copy(zeros_hbm, od_sh.at[pl.ds(s * RPT, RPT)])
    pltpu.sync_copy(zeros_hbm, id_sh.at[pl.ds(s * RPT, RPT)])
    plsc.subcore_barrier()

    def body(r, carry):
        for t in range(4):
            j = 4 * r + t
            pltpu.async_copy(ones_v, od_sh.at[sidx.at[j]], dsem[t], add=True)
            pltpu.async_copy(ones_v, id_sh.at[didx.at[j]], dsem[t], add=True)
        for t in range(4):
            pltpu.make_async_copy(ones_v, od_sh.at[sidx.at[0]],
                                  dsem[t]).wait()
            pltpu.make_async_copy(ones_v, id_sh.at[didx.at[0]],
                                  dsem[t]).wait()
        return carry

    lax.fori_loop(0, CH // 4, body, 0)
    plsc.subcore_barrier()
    pltpu.sync_copy(od_sh.at[pl.ds(s * RPT, RPT)],
                    od_out.at[c, pl.ds(s * RPT, RPT)])
    pltpu.sync_copy(id_sh.at[pl.ds(s * RPT, RPT)],
                    id_out.at[c, pl.ds(s * RPT, RPT)])


def _make_mp(F, chunk, nwin):
    params = pltpu.CompilerParams(use_tc_tiling_on_sc=False)
    nch = (CH * CHUNK) // chunk
    winch = nch // nwin
    assert winch % 4 == 0 and nch * chunk == CH * CHUNK

    @functools.partial(
        pl.kernel,
        out_type=jax.ShapeDtypeStruct((NC, NPAD, F), jnp.float32),
        mesh=_mesh,
        compiler_params=params,
        scratch_types=[
            pltpu.VMEM((winch, chunk), jnp.int32),
            pltpu.VMEM((winch, chunk), jnp.int32),
            pltpu.VMEM((2, chunk, F), jnp.float32),
            pltpu.VMEM_SHARED((NPAD, F), jnp.float32),
            [pltpu.SemaphoreType.DMA] * 2,
            [pltpu.SemaphoreType.DMA] * 2,
        ],
    )
    def _mp(h_hbm, src_hbm, dst_hbm, zeros_hbm, out_hbm,
            sidx, didx, gbuf, agg_sh, gsem, ssem):
        c = lax.axis_index("c")
        s = lax.axis_index("s")
        wid = c * NS + s
        pltpu.sync_copy(zeros_hbm, agg_sh.at[pl.ds(s * RPT, RPT)])
        plsc.subcore_barrier()

        def round_body(r, carry):
            j0 = 2 * r
            j1 = j0 + 1
            pltpu.make_async_copy(h_hbm.at[sidx.at[j0]], gbuf.at[0],
                                  gsem[0]).wait()
            pltpu.async_copy(gbuf.at[0], agg_sh.at[didx.at[j0]],
                             ssem[0], add=True)
            pltpu.make_async_copy(h_hbm.at[sidx.at[j1]], gbuf.at[1],
                                  gsem[1]).wait()
            pltpu.async_copy(gbuf.at[1], agg_sh.at[didx.at[j1]],
                             ssem[1], add=True)
            pltpu.make_async_copy(gbuf.at[0], agg_sh.at[didx.at[0]],
                                  ssem[0]).wait()

            @pl.when(j0 + 2 < winch)
            def _():
                pltpu.async_copy(h_hbm.at[sidx.at[j0 + 2]], gbuf.at[0],
                                 gsem[0])

            pltpu.make_async_copy(gbuf.at[1], agg_sh.at[didx.at[0]],
                                  ssem[1]).wait()

            @pl.when(j1 + 2 < winch)
            def _():
                pltpu.async_copy(h_hbm.at[sidx.at[j1 + 2]], gbuf.at[1],
                                 gsem[1])

            return carry

        for win in range(nwin):
            pltpu.sync_copy(src_hbm.at[wid, pl.ds(win * winch, winch)], sidx)
            pltpu.sync_copy(dst_hbm.at[wid, pl.ds(win * winch, winch)], didx)
            pltpu.async_copy(h_hbm.at[sidx.at[0]], gbuf.at[0], gsem[0])
            pltpu.async_copy(h_hbm.at[sidx.at[1]], gbuf.at[1], gsem[1])
            lax.fori_loop(0, winch // 2, round_body, 0)

        plsc.subcore_barrier()
        pltpu.sync_copy(agg_sh.at[pl.ds(s * RPT, RPT)],
                        out_hbm.at[c, pl.ds(s * RPT, RPT)])

    return _mp


_mp128 = _make_mp(HID, 128, 2)
_mp64 = _make_mp(NCLS, 128, 1)

BLK = 512


def _inv_sqrt_deg(dcol):
    return jnp.where(dcol > 0, lax.rsqrt(jnp.maximum(dcol, 1.0)), 0.0)


def _l1_kernel(x_ref, w_ref, od_ref, o_ref):
    d = od_ref[0] + od_ref[1]
    sc = _inv_sqrt_deg(d[:, 0:1])
    o_ref[...] = jnp.dot(x_ref[...] * sc, w_ref[...],
                         preferred_element_type=jnp.float32)


def _l2_kernel(agg_ref, id_ref, od_ref, b1_ref, w_ref, o_ref):
    isc = _inv_sqrt_deg((id_ref[0] + id_ref[1])[:, 0:1])
    osc = _inv_sqrt_deg((od_ref[0] + od_ref[1])[:, 0:1])
    h = (agg_ref[0] + agg_ref[1]) * isc + b1_ref[...]
    h = jnp.maximum(h, 0.0) * osc
    o_ref[...] = jnp.dot(h, w_ref[...], preferred_element_type=jnp.float32)


def _out_kernel(agg_ref, id_ref, b2_ref, o_ref):
    isc = _inv_sqrt_deg((id_ref[0] + id_ref[1])[:, 0:1])
    o_ref[...] = (agg_ref[0] + agg_ref[1]) * isc + b2_ref[...]


def kernel(inputs, edge_index, W1, b1, W2, b2):
    src = edge_index[0].astype(jnp.int32)
    dst = edge_index[1].astype(jnp.int32)
    npd = EPAD - E
    pad = N + (jnp.arange(npd, dtype=jnp.int32) % PAD_SPREAD)
    srcr = jnp.concatenate([src, pad]).reshape(NW, CH, CHUNK)
    dstr = jnp.concatenate([dst, pad]).reshape(NW, CH, CHUNK)
    ones16 = jnp.ones((CHUNK, 16), jnp.float32)
    z16 = jnp.zeros((RPT, 16), jnp.float32)
    z128 = jnp.zeros((RPT, HID), jnp.float32)
    z64 = jnp.zeros((RPT, NCLS), jnp.float32)
    xp = jnp.pad(inputs, ((0, NPAD - N), (0, 0)))

    od_p, id_p = _deg_kernel(srcr, dstr, ones16, z16)

    h1s = pl.pallas_call(
        _l1_kernel,
        grid=(NPAD // BLK,),
        in_specs=[
            pl.BlockSpec((BLK, F_IN), lambda i: (i, 0)),
            pl.BlockSpec((F_IN, HID), lambda i: (0, 0)),
            pl.BlockSpec((NC, BLK, 16), lambda i: (0, i, 0)),
        ],
        out_specs=pl.BlockSpec((BLK, HID), lambda i: (i, 0)),
        out_shape=jax.ShapeDtypeStruct((NPAD, HID), jnp.float32),
    )(xp, W1, od_p)

    agg1 = _mp128(h1s, srcr, dstr, z128)

    h2s = pl.pallas_call(
        _l2_kernel,
        grid=(NPAD // BLK,),
        in_specs=[
            pl.BlockSpec((NC, BLK, HID), lambda i: (0, i, 0)),
            pl.BlockSpec((NC, BLK, 16), lambda i: (0, i, 0)),
            pl.BlockSpec((NC, BLK, 16), lambda i: (0, i, 0)),
            pl.BlockSpec((1, HID), lambda i: (0, 0)),
            pl.BlockSpec((HID, NCLS), lambda i: (0, 0)),
        ],
        out_specs=pl.BlockSpec((BLK, NCLS), lambda i: (i, 0)),
        out_shape=jax.ShapeDtypeStruct((NPAD, NCLS), jnp.float32),
    )(agg1, id_p, od_p, b1.reshape(1, HID), W2)

    agg2 = _mp64(h2s, srcr, dstr, z64)

    OBLK = 400
    out = pl.pallas_call(
        _out_kernel,
        grid=(N // OBLK,),
        in_specs=[
            pl.BlockSpec((NC, OBLK, NCLS), lambda i: (0, i, 0)),
            pl.BlockSpec((NC, OBLK, 16), lambda i: (0, i, 0)),
            pl.BlockSpec((1, NCLS), lambda i: (0, 0)),
        ],
        out_specs=pl.BlockSpec((OBLK, NCLS), lambda i: (i, 0)),
        out_shape=jax.ShapeDtypeStruct((N, NCLS), jnp.float32),
    )(agg2, id_p, b2.reshape(1, NCLS))

    return out


# speedup vs baseline: 1.1783x; 1.1783x over previous
import functools

import jax
import jax.numpy as jnp
from jax import lax
from jax.experimental import pallas as pl
from jax.experimental.pallas import tpu as pltpu
from jax.experimental.pallas import tpu_sc as plsc

N = 10000
E = 320000
F_IN = 128
HID = 128
NCLS = 64

NC = 2
NS = 16
NW = NC * NS

CHUNK = 128
CH = 80
IDXW = 40
EPAD = NW * CH * CHUNK
NPAD = 10240
RPT = NPAD // NS
PAD_SPREAD = NPAD - N

_mesh = plsc.VectorSubcoreMesh(core_axis_name="c", subcore_axis_name="s")


@functools.partial(
    pl.kernel,
    out_type=(
        jax.ShapeDtypeStruct((NC, NPAD, 16), jnp.float32),
        jax.ShapeDtypeStruct((NC, NPAD, 16), jnp.float32),
    ),
    mesh=_mesh,
    compiler_params=pltpu.CompilerParams(use_tc_tiling_on_sc=False),
    scratch_types=[
        pltpu.VMEM((CH, CHUNK), jnp.int32),
        pltpu.VMEM((CH, CHUNK), jnp.int32),
        pltpu.VMEM((CHUNK, 16), jnp.float32),
        pltpu.VMEM_SHARED((NPAD, 16), jnp.float32),
        pltpu.VMEM_SHARED((NPAD, 16), jnp.float32),
        [pltpu.SemaphoreType.DMA] * 4,
    ],
)
def _deg_kernel(src_hbm, dst_hbm, ones_hbm, zeros_hbm, od_out, id_out,
                sidx, didx, ones_v, od_sh, id_sh, dsem):
    c = lax.axis_index("c")
    s = lax.axis_index("s")
    wid = c * NS + s
    pltpu.sync_copy(src_hbm.at[wid], sidx)
    pltpu.sync_copy(dst_hbm.at[wid], didx)
    pltpu.sync_copy(ones_hbm, ones_v)
    pltpu.sync_copy(zeros_hbm, od_sh.at[pl.ds(s * RPT, RPT)])
    pltpu.sync_copy(zeros_hbm, id_sh.at[pl.ds(s * RPT, RPT)])
    plsc.subcore_barrier()

    def body(r, carry):
        for t in range(4):
            j = 4 * r + t
            pltpu.async_copy(ones_v, od_sh.at[sidx.at[j]], dsem[t], add=True)
            pltpu.async_copy(ones_v, id_sh.at[didx.at[j]], dsem[t], add=True)
        for t in range(4):
            pltpu.make_async_copy(ones_v, od_sh.at[sidx.at[0]],
                                  dsem[t]).wait()
            pltpu.make_async_copy(ones_v, id_sh.at[didx.at[0]],
                                  dsem[t]).wait()
        return carry

    lax.fori_loop(0, CH // 4, body, 0)
    plsc.subcore_barrier()
    pltpu.sync_copy(od_sh.at[pl.ds(s * RPT, RPT)],
                    od_out.at[c, pl.ds(s * RPT, RPT)])
    pltpu.sync_copy(id_sh.at[pl.ds(s * RPT, RPT)],
                    id_out.at[c, pl.ds(s * RPT, RPT)])


def _make_mp(F, chunk, nwin):
    params = pltpu.CompilerParams(use_tc_tiling_on_sc=False)
    nch = (CH * CHUNK) // chunk
    winch = nch // nwin
    assert winch % 4 == 0 and nch * chunk == CH * CHUNK

    @functools.partial(
        pl.kernel,
        out_type=jax.ShapeDtypeStruct((NC, NPAD, F), jnp.float32),
        mesh=_mesh,
        compiler_params=params,
        scratch_types=[
            pltpu.VMEM((winch, chunk), jnp.int32),
            pltpu.VMEM((winch, chunk), jnp.int32),
            pltpu.VMEM((2, chunk, F), jnp.float32),
            pltpu.VMEM_SHARED((NPAD, F), jnp.float32),
            [pltpu.SemaphoreType.DMA] * 2,
            [pltpu.SemaphoreType.DMA] * 2,
        ],
    )
    def _mp(h_hbm, src_hbm, dst_hbm, zeros_hbm, out_hbm,
            sidx, didx, gbuf, agg_sh, gsem, ssem):
        c = lax.axis_index("c")
        s = lax.axis_index("s")
        wid = c * NS + s
        pltpu.sync_copy(zeros_hbm, agg_sh.at[pl.ds(s * RPT, RPT)])
        plsc.subcore_barrier()

        def round_body(r, carry):
            j0 = 2 * r
            j1 = j0 + 1
            pltpu.make_async_copy(h_hbm.at[sidx.at[j0]], gbuf.at[0],
                                  gsem[0]).wait()
            pltpu.sync_copy(gbuf.at[0], agg_sh.at[didx.at[j0]], add=True)

            @pl.when(j0 + 2 < winch)
            def _():
                pltpu.async_copy(h_hbm.at[sidx.at[j0 + 2]], gbuf.at[0],
                                 gsem[0])

            pltpu.make_async_copy(h_hbm.at[sidx.at[j1]], gbuf.at[1],
                                  gsem[1]).wait()
            pltpu.sync_copy(gbuf.at[1], agg_sh.at[didx.at[j1]], add=True)

            @pl.when(j1 + 2 < winch)
            def _():
                pltpu.async_copy(h_hbm.at[sidx.at[j1 + 2]], gbuf.at[1],
                                 gsem[1])

            return carry

        for win in range(nwin):
            pltpu.sync_copy(src_hbm.at[wid, pl.ds(win * winch, winch)], sidx)
            pltpu.sync_copy(dst_hbm.at[wid, pl.ds(win * winch, winch)], didx)
            pltpu.async_copy(h_hbm.at[sidx.at[0]], gbuf.at[0], gsem[0])
            pltpu.async_copy(h_hbm.at[sidx.at[1]], gbuf.at[1], gsem[1])
            lax.fori_loop(0, winch // 2, round_body, 0)

        plsc.subcore_barrier()
        pltpu.sync_copy(agg_sh.at[pl.ds(s * RPT, RPT)],
                        out_hbm.at[c, pl.ds(s * RPT, RPT)])

    return _mp


def _make_mp4(F, chunk, nwin):
    params = pltpu.CompilerParams(use_tc_tiling_on_sc=False)
    nch = (CH * CHUNK) // chunk
    winch = nch // nwin
    assert winch % 4 == 0 and nch * chunk == CH * CHUNK

    @functools.partial(
        pl.kernel,
        out_type=jax.ShapeDtypeStruct((NC, NPAD, F), jnp.float32),
        mesh=_mesh,
        compiler_params=params,
        scratch_types=[
            pltpu.VMEM((winch, chunk), jnp.int32),
            pltpu.VMEM((winch, chunk), jnp.int32),
            pltpu.VMEM((4, chunk, F), jnp.float32),
            pltpu.VMEM_SHARED((NPAD, F), jnp.float32),
            [pltpu.SemaphoreType.DMA] * 4,
            [pltpu.SemaphoreType.DMA] * 4,
        ],
    )
    def _mp(h_hbm, src_hbm, dst_hbm, zeros_hbm, out_hbm,
            sidx, didx, gbuf, agg_sh, gsem, ssem):
        c = lax.axis_index("c")
        s = lax.axis_index("s")
        wid = c * NS + s
        pltpu.sync_copy(zeros_hbm, agg_sh.at[pl.ds(s * RPT, RPT)])
        plsc.subcore_barrier()

        def round_body(r, carry):
            for t in range(4):
                j = 4 * r + t
                pltpu.make_async_copy(h_hbm.at[sidx.at[j]], gbuf.at[t],
                                      gsem[t]).wait()
                pltpu.async_copy(gbuf.at[t], agg_sh.at[didx.at[j]],
                                 ssem[t], add=True)
                tp = (t + 2) % 4

                @pl.when(j >= 2)
                def _():
                    pltpu.make_async_copy(gbuf.at[tp],
                                          agg_sh.at[didx.at[0]],
                                          ssem[tp]).wait()

                @pl.when(j + 2 < winch)
                def _():
                    pltpu.async_copy(h_hbm.at[sidx.at[j + 2]], gbuf.at[tp],
                                     gsem[tp])
            return carry

        for win in range(nwin):
            pltpu.sync_copy(src_hbm.at[wid, pl.ds(win * winch, winch)], sidx)
            pltpu.sync_copy(dst_hbm.at[wid, pl.ds(win * winch, winch)], didx)
            pltpu.async_copy(h_hbm.at[sidx.at[0]], gbuf.at[0], gsem[0])
            pltpu.async_copy(h_hbm.at[sidx.at[1]], gbuf.at[1], gsem[1])
            lax.fori_loop(0, winch // 4, round_body, 0)
            pltpu.make_async_copy(gbuf.at[2], agg_sh.at[didx.at[0]],
                                  ssem[2]).wait()
            pltpu.make_async_copy(gbuf.at[3], agg_sh.at[didx.at[0]],
                                  ssem[3]).wait()

        plsc.subcore_barrier()
        pltpu.sync_copy(agg_sh.at[pl.ds(s * RPT, RPT)],
                        out_hbm.at[c, pl.ds(s * RPT, RPT)])

    return _mp


_mp128 = _make_mp(HID, 128, 2)
_mp64 = _make_mp4(NCLS, 128, 1)

BLK = 512


def _inv_sqrt_deg(dcol):
    return jnp.where(dcol > 0, lax.rsqrt(jnp.maximum(dcol, 1.0)), 0.0)


def _l1_kernel(x_ref, w_ref, od_ref, o_ref):
    d = od_ref[0] + od_ref[1]
    sc = _inv_sqrt_deg(d[:, 0:1])
    o_ref[...] = jnp.dot(x_ref[...] * sc, w_ref[...],
                         preferred_element_type=jnp.float32)


def _l2_kernel(agg_ref, id_ref, od_ref, b1_ref, w_ref, o_ref):
    isc = _inv_sqrt_deg((id_ref[0] + id_ref[1])[:, 0:1])
    osc = _inv_sqrt_deg((od_ref[0] + od_ref[1])[:, 0:1])
    h = (agg_ref[0] + agg_ref[1]) * isc + b1_ref[...]
    h = jnp.maximum(h, 0.0) * osc
    o_ref[...] = jnp.dot(h, w_ref[...], preferred_element_type=jnp.float32)


def _out_kernel(agg_ref, id_ref, b2_ref, o_ref):
    isc = _inv_sqrt_deg((id_ref[0] + id_ref[1])[:, 0:1])
    o_ref[...] = (agg_ref[0] + agg_ref[1]) * isc + b2_ref[...]


def kernel(inputs, edge_index, W1, b1, W2, b2):
    src = edge_index[0].astype(jnp.int32)
    dst = edge_index[1].astype(jnp.int32)
    npd = EPAD - E
    pad = N + (jnp.arange(npd, dtype=jnp.int32) % PAD_SPREAD)
    srcr = jnp.concatenate([src, pad]).reshape(NW, CH, CHUNK)
    dstr = jnp.concatenate([dst, pad]).reshape(NW, CH, CHUNK)
    ones16 = jnp.ones((CHUNK, 16), jnp.float32)
    z16 = jnp.zeros((RPT, 16), jnp.float32)
    z128 = jnp.zeros((RPT, HID), jnp.float32)
    z64 = jnp.zeros((RPT, NCLS), jnp.float32)
    xp = jnp.pad(inputs, ((0, NPAD - N), (0, 0)))

    od_p, id_p = _deg_kernel(srcr, dstr, ones16, z16)

    h1s = pl.pallas_call(
        _l1_kernel,
        grid=(NPAD // BLK,),
        in_specs=[
            pl.BlockSpec((BLK, F_IN), lambda i: (i, 0)),
            pl.BlockSpec((F_IN, HID), lambda i: (0, 0)),
            pl.BlockSpec((NC, BLK, 16), lambda i: (0, i, 0)),
        ],
        out_specs=pl.BlockSpec((BLK, HID), lambda i: (i, 0)),
        out_shape=jax.ShapeDtypeStruct((NPAD, HID), jnp.float32),
    )(xp, W1, od_p)

    agg1 = _mp128(h1s, srcr, dstr, z128)

    h2s = pl.pallas_call(
        _l2_kernel,
        grid=(NPAD // BLK,),
        in_specs=[
            pl.BlockSpec((NC, BLK, HID), lambda i: (0, i, 0)),
            pl.BlockSpec((NC, BLK, 16), lambda i: (0, i, 0)),
            pl.BlockSpec((NC, BLK, 16), lambda i: (0, i, 0)),
            pl.BlockSpec((1, HID), lambda i: (0, 0)),
            pl.BlockSpec((HID, NCLS), lambda i: (0, 0)),
        ],
        out_specs=pl.BlockSpec((BLK, NCLS), lambda i: (i, 0)),
        out_shape=jax.ShapeDtypeStruct((NPAD, NCLS), jnp.float32),
    )(agg1, id_p, od_p, b1.reshape(1, HID), W2)

    agg2 = _mp64(h2s, srcr, dstr, z64)

    OBLK = 400
    out = pl.pallas_call(
        _out_kernel,
        grid=(N // OBLK,),
        in_specs=[
            pl.BlockSpec((NC, OBLK, NCLS), lambda i: (0, i, 0)),
            pl.BlockSpec((NC, OBLK, 16), lambda i: (0, i, 0)),
            pl.BlockSpec((1, NCLS), lambda i: (0, 0)),
        ],
        out_specs=pl.BlockSpec((OBLK, NCLS), lambda i: (i, 0)),
        out_shape=jax.ShapeDtypeStruct((N, NCLS), jnp.float32),
    )(agg2, id_p, b2.reshape(1, NCLS))

    return out


# speedup vs baseline: 1.2878x; 1.0929x over previous
import functools

import jax
import jax.numpy as jnp
from jax import lax
from jax.experimental import pallas as pl
from jax.experimental.pallas import tpu as pltpu
from jax.experimental.pallas import tpu_sc as plsc

N = 10000
E = 320000
F_IN = 128
HID = 128
NCLS = 64

NC = 2
NS = 16
NW = NC * NS

CHUNK = 128
CH = 80
IDXW = 40
EPAD = NW * CH * CHUNK
NPAD = 10240
RPT = NPAD // NS
PAD_SPREAD = NPAD - N

_mesh = plsc.VectorSubcoreMesh(core_axis_name="c", subcore_axis_name="s")


@functools.partial(
    pl.kernel,
    out_type=(
        jax.ShapeDtypeStruct((NW, NPAD), jnp.float32),
        jax.ShapeDtypeStruct((NW, NPAD), jnp.float32),
    ),
    mesh=_mesh,
    compiler_params=pltpu.CompilerParams(needs_layout_passes=False),
    scratch_types=[
        pltpu.VMEM((CH, CHUNK), jnp.int32),
        pltpu.VMEM((CH, CHUNK), jnp.int32),
        pltpu.VMEM((NPAD,), jnp.float32),
        pltpu.VMEM((NPAD,), jnp.float32),
    ],
)
def _deg_kernel(src_hbm, dst_hbm, zeros_hbm, od_out, id_out,
                sidx, didx, oh, ih):
    c = lax.axis_index("c")
    s = lax.axis_index("s")
    wid = c * NS + s
    pltpu.sync_copy(src_hbm.at[wid], sidx)
    pltpu.sync_copy(dst_hbm.at[wid], didx)
    pltpu.sync_copy(zeros_hbm, oh)
    pltpu.sync_copy(zeros_hbm, ih)
    ones = jnp.ones((16,), jnp.float32)

    def body(j, carry):
        def inner(k, carry2):
            sv = sidx[j, pl.ds(k * 16, 16)]
            dv = didx[j, pl.ds(k * 16, 16)]
            plsc.addupdate_scatter(oh, [sv], ones)
            plsc.addupdate_scatter(ih, [dv], ones)
            return carry2
        return lax.fori_loop(0, CHUNK // 16, inner, carry)

    lax.fori_loop(0, CH, body, 0)
    pltpu.sync_copy(oh, od_out.at[wid])
    pltpu.sync_copy(ih, id_out.at[wid])


def _make_mp(F, chunk, nwin):
    params = pltpu.CompilerParams(use_tc_tiling_on_sc=False)
    nch = (CH * CHUNK) // chunk
    winch = nch // nwin
    assert winch % 4 == 0 and nch * chunk == CH * CHUNK

    @functools.partial(
        pl.kernel,
        out_type=jax.ShapeDtypeStruct((NC, NPAD, F), jnp.float32),
        mesh=_mesh,
        compiler_params=params,
        scratch_types=[
            pltpu.VMEM((winch, chunk), jnp.int32),
            pltpu.VMEM((winch, chunk), jnp.int32),
            pltpu.VMEM((2, chunk, F), jnp.float32),
            pltpu.VMEM_SHARED((NPAD, F), jnp.float32),
            [pltpu.SemaphoreType.DMA] * 2,
            [pltpu.SemaphoreType.DMA] * 2,
        ],
    )
    def _mp(h_hbm, src_hbm, dst_hbm, zeros_hbm, out_hbm,
            sidx, didx, gbuf, agg_sh, gsem, ssem):
        c = lax.axis_index("c")
        s = lax.axis_index("s")
        wid = c * NS + s
        pltpu.sync_copy(zeros_hbm, agg_sh.at[pl.ds(s * RPT, RPT)])
        plsc.subcore_barrier()

        def round_body(r, carry):
            j0 = 2 * r
            j1 = j0 + 1
            pltpu.make_async_copy(h_hbm.at[sidx.at[j0]], gbuf.at[0],
                                  gsem[0]).wait()
            pltpu.sync_copy(gbuf.at[0], agg_sh.at[didx.at[j0]], add=True)

            @pl.when(j0 + 2 < winch)
            def _():
                pltpu.async_copy(h_hbm.at[sidx.at[j0 + 2]], gbuf.at[0],
                                 gsem[0])

            pltpu.make_async_copy(h_hbm.at[sidx.at[j1]], gbuf.at[1],
                                  gsem[1]).wait()
            pltpu.sync_copy(gbuf.at[1], agg_sh.at[didx.at[j1]], add=True)

            @pl.when(j1 + 2 < winch)
            def _():
                pltpu.async_copy(h_hbm.at[sidx.at[j1 + 2]], gbuf.at[1],
                                 gsem[1])

            return carry

        for win in range(nwin):
            pltpu.sync_copy(src_hbm.at[wid, pl.ds(win * winch, winch)], sidx)
            pltpu.sync_copy(dst_hbm.at[wid, pl.ds(win * winch, winch)], didx)
            pltpu.async_copy(h_hbm.at[sidx.at[0]], gbuf.at[0], gsem[0])
            pltpu.async_copy(h_hbm.at[sidx.at[1]], gbuf.at[1], gsem[1])
            lax.fori_loop(0, winch // 2, round_body, 0)

        plsc.subcore_barrier()
        pltpu.sync_copy(agg_sh.at[pl.ds(s * RPT, RPT)],
                        out_hbm.at[c, pl.ds(s * RPT, RPT)])

    return _mp


def _make_mp4(F, chunk, nwin):
    params = pltpu.CompilerParams(use_tc_tiling_on_sc=False)
    nch = (CH * CHUNK) // chunk
    winch = nch // nwin
    assert winch % 4 == 0 and nch * chunk == CH * CHUNK

    @functools.partial(
        pl.kernel,
        out_type=jax.ShapeDtypeStruct((NC, NPAD, F), jnp.float32),
        mesh=_mesh,
        compiler_params=params,
        scratch_types=[
            pltpu.VMEM((winch, chunk), jnp.int32),
            pltpu.VMEM((winch, chunk), jnp.int32),
            pltpu.VMEM((4, chunk, F), jnp.float32),
            pltpu.VMEM_SHARED((NPAD, F), jnp.float32),
            [pltpu.SemaphoreType.DMA] * 4,
            [pltpu.SemaphoreType.DMA] * 4,
        ],
    )
    def _mp(h_hbm, src_hbm, dst_hbm, zeros_hbm, out_hbm,
            sidx, didx, gbuf, agg_sh, gsem, ssem):
        c = lax.axis_index("c")
        s = lax.axis_index("s")
        wid = c * NS + s
        pltpu.sync_copy(zeros_hbm, agg_sh.at[pl.ds(s * RPT, RPT)])
        plsc.subcore_barrier()

        def round_body(r, carry):
            for t in range(4):
                j = 4 * r + t
                pltpu.make_async_copy(h_hbm.at[sidx.at[j]], gbuf.at[t],
                                      gsem[t]).wait()
                pltpu.async_copy(gbuf.at[t], agg_sh.at[didx.at[j]],
                                 ssem[t], add=True)
                tp = (t + 2) % 4

                @pl.when(j >= 2)
                def _():
                    pltpu.make_async_copy(gbuf.at[tp],
                                          agg_sh.at[didx.at[0]],
                                          ssem[tp]).wait()

                @pl.when(j + 2 < winch)
                def _():
                    pltpu.async_copy(h_hbm.at[sidx.at[j + 2]], gbuf.at[tp],
                                     gsem[tp])
            return carry

        for win in range(nwin):
            pltpu.sync_copy(src_hbm.at[wid, pl.ds(win * winch, winch)], sidx)
            pltpu.sync_copy(dst_hbm.at[wid, pl.ds(win * winch, winch)], didx)
            pltpu.async_copy(h_hbm.at[sidx.at[0]], gbuf.at[0], gsem[0])
            pltpu.async_copy(h_hbm.at[sidx.at[1]], gbuf.at[1], gsem[1])
            lax.fori_loop(0, winch // 4, round_body, 0)
            pltpu.make_async_copy(gbuf.at[2], agg_sh.at[didx.at[0]],
                                  ssem[2]).wait()
            pltpu.make_async_copy(gbuf.at[3], agg_sh.at[didx.at[0]],
                                  ssem[3]).wait()

        plsc.subcore_barrier()
        pltpu.sync_copy(agg_sh.at[pl.ds(s * RPT, RPT)],
                        out_hbm.at[c, pl.ds(s * RPT, RPT)])

    return _mp


_mp128 = _make_mp(HID, 128, 2)
_mp64 = _make_mp4(NCLS, 128, 1)

BLK = 512


def _inv_sqrt_deg(dcol):
    return jnp.where(dcol > 0, lax.rsqrt(jnp.maximum(dcol, 1.0)), 0.0)


def _scale_col(deg_ref):
    d = jnp.sum(deg_ref[...], axis=0, keepdims=True)
    return _inv_sqrt_deg(d).T


def _l1_kernel(x_ref, w_ref, od_ref, o_ref):
    o_ref[...] = jnp.dot(x_ref[...] * _scale_col(od_ref), w_ref[...],
                         preferred_element_type=jnp.float32)


def _l2_kernel(agg_ref, id_ref, od_ref, b1_ref, w_ref, o_ref):
    h = (agg_ref[0] + agg_ref[1]) * _scale_col(id_ref) + b1_ref[...]
    h = jnp.maximum(h, 0.0) * _scale_col(od_ref)
    o_ref[...] = jnp.dot(h, w_ref[...], preferred_element_type=jnp.float32)


def _out_kernel(agg_ref, id_ref, b2_ref, o_ref):
    o_ref[...] = (agg_ref[0] + agg_ref[1]) * _scale_col(id_ref) + b2_ref[...]


def kernel(inputs, edge_index, W1, b1, W2, b2):
    src = edge_index[0].astype(jnp.int32)
    dst = edge_index[1].astype(jnp.int32)
    npd = EPAD - E
    pad = N + (jnp.arange(npd, dtype=jnp.int32) % PAD_SPREAD)
    srcr = jnp.concatenate([src, pad]).reshape(NW, CH, CHUNK)
    dstr = jnp.concatenate([dst, pad]).reshape(NW, CH, CHUNK)
    z128 = jnp.zeros((RPT, HID), jnp.float32)
    z64 = jnp.zeros((RPT, NCLS), jnp.float32)
    z1d = jnp.zeros((NPAD,), jnp.float32)

    od_p, id_p = _deg_kernel(srcr, dstr, z1d)

    h1s = pl.pallas_call(
        _l1_kernel,
        grid=(NPAD // BLK,),
        in_specs=[
            pl.BlockSpec((BLK, F_IN), lambda i: (i, 0)),
            pl.BlockSpec((F_IN, HID), lambda i: (0, 0)),
            pl.BlockSpec((NW, BLK), lambda i: (0, i)),
        ],
        out_specs=pl.BlockSpec((BLK, HID), lambda i: (i, 0)),
        out_shape=jax.ShapeDtypeStruct((NPAD, HID), jnp.float32),
    )(inputs, W1, od_p)

    agg1 = _mp128(h1s, srcr, dstr, z128)

    h2s = pl.pallas_call(
        _l2_kernel,
        grid=(NPAD // BLK,),
        in_specs=[
            pl.BlockSpec((NC, BLK, HID), lambda i: (0, i, 0)),
            pl.BlockSpec((NW, BLK), lambda i: (0, i)),
            pl.BlockSpec((NW, BLK), lambda i: (0, i)),
            pl.BlockSpec((1, HID), lambda i: (0, 0)),
            pl.BlockSpec((HID, NCLS), lambda i: (0, 0)),
        ],
        out_specs=pl.BlockSpec((BLK, NCLS), lambda i: (i, 0)),
        out_shape=jax.ShapeDtypeStruct((NPAD, NCLS), jnp.float32),
    )(agg1, id_p, od_p, b1.reshape(1, HID), W2)

    agg2 = _mp64(h2s, srcr, dstr, z64)

    out = pl.pallas_call(
        _out_kernel,
        grid=(NPAD // BLK,),
        in_specs=[
            pl.BlockSpec((NC, BLK, NCLS), lambda i: (0, i, 0)),
            pl.BlockSpec((NW, BLK), lambda i: (0, i)),
            pl.BlockSpec((1, NCLS), lambda i: (0, 0)),
        ],
        out_specs=pl.BlockSpec((BLK, NCLS), lambda i: (i, 0)),
        out_shape=jax.ShapeDtypeStruct((NPAD, NCLS), jnp.float32),
    )(agg2, id_p, b2.reshape(1, NCLS))

    return out[:N]
